# Initial kernel scaffold; baseline (speedup 1.0000x reference)
import jax
import jax.numpy as jnp
from jax.experimental import pallas as pl


def kernel(x, edge_index, edge_type, W_basis, W_comp):
    raise NotImplementedError("write your pallas kernel here")


# speedup vs baseline: 26.7807x; 26.7807x over previous
import functools

import jax
import jax.numpy as jnp
from jax import lax
from jax.experimental import pallas as pl
from jax.experimental.pallas import tpu as pltpu
from jax.experimental.pallas import tpu_sc as plsc

NC = 2
NS = 16
LANES = 16


def _project_kernel(wc_ref, wb_ref, x_ref, out_ref):
    r = pl.program_id(0)
    w = (wc_ref[r, 0] * wb_ref[0]
         + wc_ref[r, 1] * wb_ref[1]
         + wc_ref[r, 2] * wb_ref[2]
         + wc_ref[r, 3] * wb_ref[3])
    out_ref[0] = jnp.dot(x_ref[...], w, preferred_element_type=jnp.float32)


def _finalize_kernel(p_ref, out_ref):
    out_ref[...] = jnp.maximum(p_ref[0] + p_ref[1], 0.0)


def _sc_edge_kernel(n_nodes, n_edges, d, chunk,
                    pre_hbm, src_hbm, dst_hbm, typ_hbm, zeros_hbm, part_hbm,
                    srcv, typv, dstv, idxv, rows, acc, sem):
    c = lax.axis_index("c")
    s = lax.axis_index("s")
    wid = c * NS + s

    edges_per_tile = n_edges // (NC * NS)
    n_chunks = edges_per_tile // chunk

    rblk = 80
    n_rblk = n_nodes // rblk
    rblk_iters = (n_rblk + NS - 1) // NS

    def zero_body(it, _):
        j = it * NS + s

        @pl.when(j < n_rblk)
        def _():
            pltpu.sync_copy(zeros_hbm, acc.at[pl.ds(j * rblk, rblk)])
        return ()

    lax.fori_loop(0, rblk_iters, zero_body, ())
    plsc.subcore_barrier()

    base = wid * edges_per_tile

    def body(ch, _):
        off = base + ch * chunk
        pltpu.sync_copy(src_hbm.at[pl.ds(off, chunk)], srcv)
        pltpu.sync_copy(typ_hbm.at[pl.ds(off, chunk)], typv)
        pltpu.sync_copy(dst_hbm.at[pl.ds(off, chunk)], dstv)
        for i in range(chunk // LANES):
            sl = pl.ds(i * LANES, LANES)
            idxv[sl] = typv[sl] * n_nodes + srcv[sl]
        pltpu.async_copy(pre_hbm.at[idxv], rows, sem).wait()
        pltpu.sync_copy(rows, acc.at[dstv], add=True)
        return ()

    lax.fori_loop(0, n_chunks, body, (), unroll=False)

    plsc.subcore_barrier()

    def out_body(it, _):
        j = it * NS + s

        @pl.when(j < n_rblk)
        def _():
            pltpu.sync_copy(acc.at[pl.ds(j * rblk, rblk)],
                            part_hbm.at[c, pl.ds(j * rblk, rblk)])
        return ()

    lax.fori_loop(0, rblk_iters, out_body, ())


def kernel(x, edge_index, edge_type, W_basis, W_comp):
    n_nodes, d_in = x.shape
    n_basis, _, d_out = W_basis.shape
    n_rel = W_comp.shape[0]
    n_edges = edge_type.shape[0]

    src = edge_index[0].astype(jnp.int32)
    dst = edge_index[1].astype(jnp.int32)
    typ = edge_type.astype(jnp.int32)

    bn = 2000
    nb = n_nodes // bn
    pre = pl.pallas_call(
        _project_kernel,
        grid=(n_rel, nb),
        in_specs=[
            pl.BlockSpec(memory_space=pltpu.SMEM),
            pl.BlockSpec((n_basis, d_in, d_out), lambda r, b: (0, 0, 0)),
            pl.BlockSpec((bn, d_in), lambda r, b: (b, 0)),
        ],
        out_specs=pl.BlockSpec((1, bn, d_out), lambda r, b: (r, b, 0)),
        out_shape=jax.ShapeDtypeStruct((n_rel, n_nodes, d_out), jnp.float32),
    )(W_comp, W_basis, x)
    pre_flat = pre.reshape(n_rel * n_nodes, d_out)

    chunk = 80
    zeros = jnp.zeros((80, d_out), jnp.float32)

    mesh = plsc.VectorSubcoreMesh(core_axis_name="c", subcore_axis_name="s")
    sc_fn = pl.kernel(
        functools.partial(_sc_edge_kernel, n_nodes, n_edges, d_out, chunk),
        out_type=jax.ShapeDtypeStruct((NC, n_nodes, d_out), jnp.float32),
        mesh=mesh,
        scratch_types=[
            pltpu.VMEM((chunk,), jnp.int32),
            pltpu.VMEM((chunk,), jnp.int32),
            pltpu.VMEM((chunk,), jnp.int32),
            pltpu.VMEM((chunk,), jnp.int32),
            pltpu.VMEM((chunk, d_out), jnp.float32),
            pltpu.VMEM_SHARED((n_nodes, d_out), jnp.float32),
            pltpu.SemaphoreType.DMA,
        ],
    )
    partials = sc_fn(pre_flat, src, dst, typ, zeros)

    out = pl.pallas_call(
        _finalize_kernel,
        grid=(nb,),
        in_specs=[pl.BlockSpec((NC, bn, d_out), lambda b: (0, b, 0))],
        out_specs=pl.BlockSpec((bn, d_out), lambda b: (b, 0)),
        out_shape=jax.ShapeDtypeStruct((n_nodes, d_out), jnp.float32),
    )(partials)
    return out
